# Initial kernel scaffold; baseline (speedup 1.0000x reference)
import functools

import jax
import jax.numpy as jnp
from jax import lax
from jax.experimental import pallas as pl
from jax.experimental.pallas import tpu as pltpu
from jax.experimental.pallas import tpu_sc as plsc

VOCAB = 100000
EMB_DIM = 64
PART_DIM = 128

NUM_SC_CORES = 2
NUM_SC_SUBCORES = 16
NUM_WORKERS = NUM_SC_CORES * NUM_SC_SUBCORES


def _sc_gather(idx_all, table, chunk):
    n = idx_all.shape[0]
    per_w = n // NUM_WORKERS
    n_chunks = per_w // chunk
    assert per_w % chunk == 0 and chunk % 8 == 0

    mesh = plsc.VectorSubcoreMesh(core_axis_name="c", subcore_axis_name="s")

    @functools.partial(
        pl.kernel,
        mesh=mesh,
        out_type=jax.ShapeDtypeStruct((n, EMB_DIM), table.dtype),
        scratch_types=[
            pltpu.VMEM((chunk,), jnp.int32),
            pltpu.VMEM((chunk, EMB_DIM), table.dtype),
            pltpu.SemaphoreType.DMA,
        ],
    )
    def gather_k(idx_hbm, table_hbm, out_hbm, idx_v, rows_v, sem):
        wid = lax.axis_index("s") * NUM_SC_CORES + lax.axis_index("c")
        w_base = wid * per_w

        def body(g, carry):
            base = w_base + g * chunk
            pltpu.sync_copy(idx_hbm.at[pl.ds(base, chunk)], idx_v)
            pltpu.async_copy(table_hbm.at[idx_v], rows_v, sem).wait()
            pltpu.sync_copy(rows_v, out_hbm.at[pl.ds(base, chunk)])
            return carry

        lax.fori_loop(0, n_chunks, body, 0, unroll=False)

    return gather_k(idx_all, table)


def _tc_encode(embeds, W, b2d, blk):
    two_n = embeds.shape[0]
    n = two_n // 2
    n_blocks = n // blk

    def mm_kernel(a_ref, m_ref, w_ref, b_ref, o_ref):
        w = w_ref[...]
        acc = lax.dot_general(
            a_ref[...], w[:, :EMB_DIM],
            (((1,), (1,)), ((), ())),
            preferred_element_type=jnp.float32,
        )
        acc += lax.dot_general(
            m_ref[...], w[:, EMB_DIM:],
            (((1,), (1,)), ((), ())),
            preferred_element_type=jnp.float32,
        )
        o_ref[...] = jnp.maximum(acc + b_ref[...], 0.0)

    return pl.pallas_call(
        mm_kernel,
        grid=(n_blocks,),
        in_specs=[
            pl.BlockSpec((blk, EMB_DIM), lambda i: (i, 0)),
            pl.BlockSpec((blk, EMB_DIM), lambda i: (i + n_blocks, 0)),
            pl.BlockSpec((PART_DIM, 2 * EMB_DIM), lambda i: (0, 0)),
            pl.BlockSpec((1, PART_DIM), lambda i: (0, 0)),
        ],
        out_specs=pl.BlockSpec((blk, PART_DIM), lambda i: (i, 0)),
        out_shape=jax.ShapeDtypeStruct((n, PART_DIM), jnp.float32),
    )(embeds, embeds, W, b2d)


def kernel(parts, aff_table, mat_table, W, b):
    B, L, _ = parts.shape
    n = B * L
    idx_all = jnp.concatenate(
        [parts[:, :, 0].reshape(-1), parts[:, :, 1].reshape(-1)]
    ).astype(jnp.int32)
    embeds = _sc_gather(idx_all, aff_table, chunk=1280)
    out = _tc_encode(embeds, W, b.reshape(1, PART_DIM), blk=2048)
    return out.reshape(B, L, PART_DIM)


# speedup vs baseline: 3.2499x; 3.2499x over previous
import functools

import jax
import jax.numpy as jnp
from jax import lax
from jax.experimental import pallas as pl
from jax.experimental.pallas import tpu as pltpu
from jax.experimental.pallas import tpu_sc as plsc

EMB_DIM = 64
PART_DIM = 128

NUM_SC_CORES = 2
NUM_SC_SUBCORES = 16
NUM_WORKERS = NUM_SC_CORES * NUM_SC_SUBCORES
LANES = 16


def _tc_project(table, w1, w2, b2d, blk):
    vocab = table.shape[0]
    nb = vocab // blk

    def proj_kernel(t_ref, w1_ref, w2_ref, b_ref, o1_ref, o2_ref):
        tb = t_ref[...]
        acc1 = lax.dot_general(
            tb, w1_ref[...], (((1,), (1,)), ((), ())),
            preferred_element_type=jnp.float32,
        )
        o1_ref[...] = acc1 + b_ref[...]
        o2_ref[...] = lax.dot_general(
            tb, w2_ref[...], (((1,), (1,)), ((), ())),
            preferred_element_type=jnp.float32,
        )

    return pl.pallas_call(
        proj_kernel,
        grid=(nb,),
        in_specs=[
            pl.BlockSpec((blk, EMB_DIM), lambda i: (i, 0)),
            pl.BlockSpec((PART_DIM, EMB_DIM), lambda i: (0, 0)),
            pl.BlockSpec((PART_DIM, EMB_DIM), lambda i: (0, 0)),
            pl.BlockSpec((1, PART_DIM), lambda i: (0, 0)),
        ],
        out_specs=[
            pl.BlockSpec((blk, PART_DIM), lambda i: (i, 0)),
            pl.BlockSpec((blk, PART_DIM), lambda i: (i, 0)),
        ],
        out_shape=[
            jax.ShapeDtypeStruct((vocab, PART_DIM), jnp.float32),
            jax.ShapeDtypeStruct((vocab, PART_DIM), jnp.float32),
        ],
    )(table, w1, w2, b2d)


def _sc_gather_combine(idx_a, idx_m, t1, t2, chunk):
    n = idx_a.shape[0]
    per_w = n // NUM_WORKERS
    n_chunks = per_w // chunk
    assert per_w % chunk == 0 and chunk % 8 == 0

    mesh = plsc.VectorSubcoreMesh(core_axis_name="c", subcore_axis_name="s")

    @functools.partial(
        pl.kernel,
        mesh=mesh,
        out_type=jax.ShapeDtypeStruct((n, PART_DIM), jnp.float32),
        scratch_types=[
            pltpu.VMEM((chunk,), jnp.int32),
            pltpu.VMEM((chunk,), jnp.int32),
            pltpu.VMEM((chunk, PART_DIM), jnp.float32),
            pltpu.VMEM((chunk, PART_DIM), jnp.float32),
            pltpu.SemaphoreType.DMA,
            pltpu.SemaphoreType.DMA,
        ],
    )
    def gather_k(idxa_hbm, idxm_hbm, t1_hbm, t2_hbm, out_hbm,
                 idxa_v, idxm_v, rows1_v, rows2_v, sem1, sem2):
        wid = lax.axis_index("s") * NUM_SC_CORES + lax.axis_index("c")
        w_base = wid * per_w

        def body(g, carry):
            base = w_base + g * chunk
            pltpu.sync_copy(idxa_hbm.at[pl.ds(base, chunk)], idxa_v)
            pltpu.sync_copy(idxm_hbm.at[pl.ds(base, chunk)], idxm_v)
            c1 = pltpu.async_copy(t1_hbm.at[idxa_v], rows1_v, sem1)
            c2 = pltpu.async_copy(t2_hbm.at[idxm_v], rows2_v, sem2)
            c1.wait()
            c2.wait()

            def row_body(j, c2):
                for c in range(PART_DIM // LANES):
                    sl = pl.ds(c * LANES, LANES)
                    v = rows1_v[j, sl] + rows2_v[j, sl]
                    rows1_v[j, sl] = jnp.maximum(v, 0.0)
                return c2

            lax.fori_loop(0, chunk, row_body, 0, unroll=False)
            pltpu.sync_copy(rows1_v, out_hbm.at[pl.ds(base, chunk)])
            return carry

        lax.fori_loop(0, n_chunks, body, 0, unroll=False)

    return gather_k(idx_a, idx_m, t1, t2)


def kernel(parts, aff_table, mat_table, W, b):
    B, L, _ = parts.shape
    idx_a = parts[:, :, 0].reshape(-1).astype(jnp.int32)
    idx_m = parts[:, :, 1].reshape(-1).astype(jnp.int32)
    w1 = W[:, :EMB_DIM]
    w2 = W[:, EMB_DIM:]
    t1, t2 = _tc_project(aff_table, w1, w2, b.reshape(1, PART_DIM), blk=2000)
    out = _sc_gather_combine(idx_a, idx_m, t1, t2, chunk=320)
    return out.reshape(B, L, PART_DIM)


# speedup vs baseline: 6.0887x; 1.8735x over previous
import functools

import jax
import jax.numpy as jnp
from jax import lax
from jax.experimental import pallas as pl
from jax.experimental.pallas import tpu as pltpu
from jax.experimental.pallas import tpu_sc as plsc

EMB_DIM = 64
PART_DIM = 128

NUM_SC_CORES = 2
NUM_SC_SUBCORES = 16
NUM_WORKERS = NUM_SC_CORES * NUM_SC_SUBCORES
LANES = 16


def _tc_project(table_t, w1, w2, b2d, blk):
    vocab = table_t.shape[1]
    nb = (vocab + blk - 1) // blk

    def proj_kernel(t_ref, w1_ref, w2_ref, b_ref, o1_ref, o2_ref):
        tb = t_ref[...]
        acc1 = lax.dot_general(
            tb, w1_ref[...], (((0,), (1,)), ((), ())),
            preferred_element_type=jnp.float32,
        )
        o1_ref[...] = acc1 + b_ref[...]
        o2_ref[...] = lax.dot_general(
            tb, w2_ref[...], (((0,), (1,)), ((), ())),
            preferred_element_type=jnp.float32,
        )

    return pl.pallas_call(
        proj_kernel,
        grid=(nb,),
        in_specs=[
            pl.BlockSpec((EMB_DIM, blk), lambda i: (0, i)),
            pl.BlockSpec((PART_DIM, EMB_DIM), lambda i: (0, 0)),
            pl.BlockSpec((PART_DIM, EMB_DIM), lambda i: (0, 0)),
            pl.BlockSpec((1, PART_DIM), lambda i: (0, 0)),
        ],
        out_specs=[
            pl.BlockSpec((blk, PART_DIM), lambda i: (i, 0)),
            pl.BlockSpec((blk, PART_DIM), lambda i: (i, 0)),
        ],
        out_shape=[
            jax.ShapeDtypeStruct((vocab, PART_DIM), jnp.float32),
            jax.ShapeDtypeStruct((vocab, PART_DIM), jnp.float32),
        ],
    )(table_t, w1, w2, b2d)


def _sc_gather_combine(idx_a, idx_m, t1, t2, chunk):
    n = idx_a.shape[0]
    per_w = n // NUM_WORKERS
    n_chunks = per_w // chunk
    assert per_w % chunk == 0 and chunk % 8 == 0

    mesh = plsc.VectorSubcoreMesh(core_axis_name="c", subcore_axis_name="s")

    @functools.partial(
        pl.kernel,
        mesh=mesh,
        out_type=jax.ShapeDtypeStruct((n, PART_DIM), jnp.float32),
        scratch_types=[
            pltpu.VMEM((chunk,), jnp.int32),
            pltpu.VMEM((chunk,), jnp.int32),
            pltpu.VMEM((chunk, PART_DIM), jnp.float32),
            pltpu.VMEM((chunk, PART_DIM), jnp.float32),
            pltpu.SemaphoreType.DMA,
            pltpu.SemaphoreType.DMA,
        ],
    )
    def gather_k(idxa_hbm, idxm_hbm, t1_hbm, t2_hbm, out_hbm,
                 idxa_v, idxm_v, rows1_v, rows2_v, sem1, sem2):
        wid = lax.axis_index("s") * NUM_SC_CORES + lax.axis_index("c")
        w_base = wid * per_w

        def body(g, carry):
            base = w_base + g * chunk
            pltpu.sync_copy(idxa_hbm.at[pl.ds(base, chunk)], idxa_v)
            pltpu.sync_copy(idxm_hbm.at[pl.ds(base, chunk)], idxm_v)
            c1 = pltpu.async_copy(t1_hbm.at[idxa_v], rows1_v, sem1)
            c2 = pltpu.async_copy(t2_hbm.at[idxm_v], rows2_v, sem2)
            c1.wait()
            c2.wait()

            def row_body(j, c2):
                for c in range(PART_DIM // LANES):
                    sl = pl.ds(c * LANES, LANES)
                    v = rows1_v[j, sl] + rows2_v[j, sl]
                    rows1_v[j, sl] = jnp.maximum(v, 0.0)
                return c2

            lax.fori_loop(0, chunk, row_body, 0, unroll=False)
            pltpu.sync_copy(rows1_v, out_hbm.at[pl.ds(base, chunk)])
            return carry

        lax.fori_loop(0, n_chunks, body, 0, unroll=False)

    return gather_k(idx_a, idx_m, t1, t2)


def kernel(parts, aff_table, mat_table, W, b):
    B, L, _ = parts.shape
    pt = jnp.transpose(parts, (1, 2, 0)).astype(jnp.int32)
    idx_a = pt[:, 0, :].reshape(-1)
    idx_m = pt[:, 1, :].reshape(-1)
    w1 = W[:, :EMB_DIM]
    w2 = W[:, EMB_DIM:]
    t1, t2 = _tc_project(aff_table.T, w1, w2, b.reshape(1, PART_DIM),
                         blk=2048)
    out_t = _sc_gather_combine(idx_a, idx_m, t1, t2, chunk=320)
    return jnp.transpose(out_t.reshape(L, B, PART_DIM), (1, 0, 2))


# speedup vs baseline: 8.3887x; 1.3777x over previous
import functools

import jax
import jax.numpy as jnp
from jax import lax
from jax.experimental import pallas as pl
from jax.experimental.pallas import tpu as pltpu
from jax.experimental.pallas import tpu_sc as plsc

EMB_DIM = 64
PART_DIM = 128

NUM_SC_CORES = 2
NUM_SC_SUBCORES = 16
NUM_WORKERS = NUM_SC_CORES * NUM_SC_SUBCORES
LANES = 16


def _tc_project(table_t, w1, w2, b2d, blk):
    vocab = table_t.shape[1]
    nb = (vocab + blk - 1) // blk

    def proj_kernel(t_ref, w1_ref, w2_ref, b_ref, o1_ref, o2_ref):
        tb = t_ref[...]
        acc1 = lax.dot_general(
            tb, w1_ref[...], (((0,), (1,)), ((), ())),
            preferred_element_type=jnp.float32,
        )
        o1_ref[...] = acc1 + b_ref[...]
        o2_ref[...] = lax.dot_general(
            tb, w2_ref[...], (((0,), (1,)), ((), ())),
            preferred_element_type=jnp.float32,
        )

    return pl.pallas_call(
        proj_kernel,
        grid=(nb,),
        in_specs=[
            pl.BlockSpec((EMB_DIM, blk), lambda i: (0, i)),
            pl.BlockSpec((PART_DIM, EMB_DIM), lambda i: (0, 0)),
            pl.BlockSpec((PART_DIM, EMB_DIM), lambda i: (0, 0)),
            pl.BlockSpec((1, PART_DIM), lambda i: (0, 0)),
        ],
        out_specs=[
            pl.BlockSpec((blk, PART_DIM), lambda i: (i, 0)),
            pl.BlockSpec((blk, PART_DIM), lambda i: (i, 0)),
        ],
        out_shape=[
            jax.ShapeDtypeStruct((vocab, PART_DIM), jnp.float32),
            jax.ShapeDtypeStruct((vocab, PART_DIM), jnp.float32),
        ],
    )(table_t, w1, w2, b2d)


def _sc_gather_combine(idx_a, idx_m, t1, t2, chunk):
    n = idx_a.shape[0]
    per_w = n // NUM_WORKERS
    n_chunks = per_w // chunk
    assert per_w % chunk == 0 and chunk % 8 == 0 and n_chunks % 2 == 0

    mesh = plsc.VectorSubcoreMesh(core_axis_name="c", subcore_axis_name="s")

    row_buf = lambda: pltpu.VMEM((chunk, PART_DIM), jnp.float32)
    idx_buf = lambda: pltpu.VMEM((chunk,), jnp.int32)

    @functools.partial(
        pl.kernel,
        mesh=mesh,
        out_type=jax.ShapeDtypeStruct((n, PART_DIM), jnp.float32),
        scratch_types=[
            [idx_buf(), idx_buf()],
            [idx_buf(), idx_buf()],
            [row_buf(), row_buf()],
            [row_buf(), row_buf()],
            [row_buf(), row_buf()],
            [pltpu.SemaphoreType.DMA] * 2,
            [pltpu.SemaphoreType.DMA] * 2,
            [pltpu.SemaphoreType.DMA] * 2,
        ],
    )
    def gather_k(idxa_hbm, idxm_hbm, t1_hbm, t2_hbm, out_hbm,
                 idxa_v, idxm_v, rows1_v, rows2_v, out_v, gs1, gs2, os):
        wid = lax.axis_index("s") * NUM_SC_CORES + lax.axis_index("c")
        w_base = wid * per_w

        def fire(g, s):
            base = w_base + g * chunk
            pltpu.sync_copy(idxa_hbm.at[pl.ds(base, chunk)], idxa_v[s])
            pltpu.sync_copy(idxm_hbm.at[pl.ds(base, chunk)], idxm_v[s])
            pltpu.async_copy(t1_hbm.at[idxa_v[s]], rows1_v[s], gs1[s])
            pltpu.async_copy(t2_hbm.at[idxm_v[s]], rows2_v[s], gs2[s])

        def handle(g, s):
            @pl.when(g + 1 < n_chunks)
            def _():
                fire(g + 1, s ^ 1)

            pltpu.make_async_copy(t1_hbm.at[idxa_v[s]], rows1_v[s],
                                  gs1[s]).wait()
            pltpu.make_async_copy(t2_hbm.at[idxm_v[s]], rows2_v[s],
                                  gs2[s]).wait()

            @pl.when(g >= 2)
            def _():
                prev = w_base + (g - 2) * chunk
                pltpu.make_async_copy(
                    out_v[s], out_hbm.at[pl.ds(prev, chunk)], os[s]).wait()

            def row_body(j, carry):
                for c in range(PART_DIM // LANES):
                    sl = pl.ds(c * LANES, LANES)
                    v = rows1_v[s][j, sl] + rows2_v[s][j, sl]
                    out_v[s][j, sl] = jnp.maximum(v, 0.0)
                return carry

            lax.fori_loop(0, chunk, row_body, 0, unroll=False)
            base = w_base + g * chunk
            pltpu.async_copy(out_v[s], out_hbm.at[pl.ds(base, chunk)], os[s])

        fire(0, 0)

        def body(i, carry):
            handle(2 * i, 0)
            handle(2 * i + 1, 1)
            return carry

        lax.fori_loop(0, n_chunks // 2, body, 0, unroll=False)
        for s in (0, 1):
            last = w_base + (n_chunks - 2 + s) * chunk
            pltpu.make_async_copy(
                out_v[s], out_hbm.at[pl.ds(last, chunk)], os[s]).wait()

    return gather_k(idx_a, idx_m, t1, t2)


def kernel(parts, aff_table, mat_table, W, b):
    B, L, _ = parts.shape
    pt = jnp.transpose(parts, (1, 2, 0)).astype(jnp.int32)
    idx_a = pt[:, 0, :].reshape(-1)
    idx_m = pt[:, 1, :].reshape(-1)
    w1 = W[:, :EMB_DIM]
    w2 = W[:, EMB_DIM:]
    t1, t2 = _tc_project(aff_table.T, w1, w2, b.reshape(1, PART_DIM),
                         blk=2048)
    out_t = _sc_gather_combine(idx_a, idx_m, t1, t2, chunk=160)
    return jnp.transpose(out_t.reshape(L, B, PART_DIM), (1, 0, 2))
